# Initial kernel scaffold; baseline (speedup 1.0000x reference)
import functools

import jax
import jax.numpy as jnp
from jax import lax
from jax.experimental import pallas as pl
from jax.experimental.pallas import tpu as pltpu
from jax.experimental.pallas import tpu_sc as plsc

FIELD_SIZE = 100000
NUM_FIELDS = 26
EMBED_DIM = 16
BATCH = 16384
TOTAL_IDX = BATCH * NUM_FIELDS


@functools.cache
def _make_sc_gather(num_cores, num_subcores):
    nw = num_cores * num_subcores
    per_w = TOTAL_IDX // nw
    n_chunks = 8
    ch = per_w // n_chunks
    mesh = plsc.VectorSubcoreMesh(core_axis_name="c", subcore_axis_name="s")

    @functools.partial(
        pl.kernel,
        mesh=mesh,
        out_type=[
            jax.ShapeDtypeStruct((TOTAL_IDX, EMBED_DIM), jnp.float32),
            jax.ShapeDtypeStruct((TOTAL_IDX, 1), jnp.float32),
        ],
        scratch_types=[
            pltpu.VMEM((ch,), jnp.int32),
            pltpu.VMEM((ch, EMBED_DIM), jnp.float32),
            pltpu.VMEM((ch, 1), jnp.float32),
            pltpu.SemaphoreType.DMA,
            pltpu.SemaphoreType.DMA,
        ],
    )
    def sc_gather(emb_hbm, lin_hbm, idx_hbm, emb_out, lin_out,
                  idx_v, rows_v, lin_v, sem_e, sem_l):
        wid = lax.axis_index("s") * num_cores + lax.axis_index("c")
        for i in range(n_chunks):
            base = wid * per_w + i * ch
            pltpu.sync_copy(idx_hbm.at[pl.ds(base, ch)], idx_v)
            cp_e = pltpu.async_copy(emb_hbm.at[idx_v], rows_v, sem_e)
            cp_l = pltpu.async_copy(lin_hbm.at[idx_v], lin_v, sem_l)
            cp_e.wait()
            cp_l.wait()
            pltpu.sync_copy(rows_v, emb_out.at[pl.ds(base, ch)])
            pltpu.sync_copy(lin_v, lin_out.at[pl.ds(base, ch)])

    return sc_gather


def _mlp_body(emb_ref, lin_ref, w1_ref, b1_ref, w2_ref, b2_ref, w3_ref,
              bias_ref, out_ref):
    e = emb_ref[...]
    h = jnp.dot(e, w1_ref[...], preferred_element_type=jnp.float32)
    h = jnp.maximum(h + b1_ref[...], 0.0)
    h = jnp.dot(h, w2_ref[...], preferred_element_type=jnp.float32)
    h = jnp.maximum(h + b2_ref[...], 0.0)
    deep = jnp.dot(h, w3_ref[...], preferred_element_type=jnp.float32)
    wide = jnp.sum(lin_ref[...], axis=1, keepdims=True)
    z = deep + wide + bias_ref[...]
    out_ref[...] = 1.0 / (1.0 + jnp.exp(-z))


def _mlp(emb_x, lin_x, w1, b1, w2, b2, w3, bias):
    bt = 2048
    in_dim = NUM_FIELDS * EMBED_DIM
    h1, h2 = w1.shape[1], w2.shape[1]
    grid = (BATCH // bt,)
    return pl.pallas_call(
        _mlp_body,
        grid=grid,
        in_specs=[
            pl.BlockSpec((bt, in_dim), lambda i: (i, 0)),
            pl.BlockSpec((bt, NUM_FIELDS), lambda i: (i, 0)),
            pl.BlockSpec((in_dim, h1), lambda i: (0, 0)),
            pl.BlockSpec((1, h1), lambda i: (0, 0)),
            pl.BlockSpec((h1, h2), lambda i: (0, 0)),
            pl.BlockSpec((1, h2), lambda i: (0, 0)),
            pl.BlockSpec((h2, 1), lambda i: (0, 0)),
            pl.BlockSpec((1, 1), lambda i: (0, 0)),
        ],
        out_specs=pl.BlockSpec((bt, 1), lambda i: (i, 0)),
        out_shape=jax.ShapeDtypeStruct((BATCH, 1), jnp.float32),
    )(emb_x, lin_x, w1, b1, w2, b2, w3, bias)


def kernel(x, emb_table, lin_w, lin_b, w1, b1, w2, b2, w3, b3):
    offsets = (jnp.arange(NUM_FIELDS, dtype=x.dtype) * FIELD_SIZE)[None, :]
    idx = (x + offsets).reshape(-1)

    info = plsc.get_sparse_core_info()
    sc_gather = _make_sc_gather(info.num_cores, info.num_subcores)
    emb_rows, lin_rows = sc_gather(emb_table, lin_w, idx)

    emb_x = emb_rows.reshape(BATCH, NUM_FIELDS * EMBED_DIM)
    lin_x = lin_rows.reshape(BATCH, NUM_FIELDS)
    bias = (lin_b + b3).reshape(1, 1)
    out = _mlp(emb_x, lin_x, w1, b1.reshape(1, -1), w2, b2.reshape(1, -1),
               w3, bias)
    return out[:, 0]


# speedup vs baseline: 7.2636x; 7.2636x over previous
import functools

import jax
import jax.numpy as jnp
from jax import lax
from jax.experimental import pallas as pl
from jax.experimental.pallas import tpu as pltpu
from jax.experimental.pallas import tpu_sc as plsc

FIELD_SIZE = 100000
NUM_FIELDS = 26
EMBED_DIM = 16
BATCH = 16384
TOTAL_IDX = BATCH * NUM_FIELDS


IDX_COLS = 128
IDX_ROWS = TOTAL_IDX // IDX_COLS


@functools.cache
def _make_sc_gather(num_cores, num_subcores):
    nw = num_cores * num_subcores
    rows_w = IDX_ROWS // nw
    n_chunks = 8
    chr_ = rows_w // n_chunks
    ch = chr_ * IDX_COLS
    mesh = plsc.VectorSubcoreMesh(core_axis_name="c", subcore_axis_name="s")

    @functools.partial(
        pl.kernel,
        mesh=mesh,
        out_type=[
            jax.ShapeDtypeStruct((TOTAL_IDX, EMBED_DIM), jnp.float32),
            jax.ShapeDtypeStruct((TOTAL_IDX,), jnp.float32),
        ],
        scratch_types=[
            pltpu.VMEM((chr_, IDX_COLS), jnp.int32),
            pltpu.VMEM((ch, EMBED_DIM), jnp.float32),
            pltpu.VMEM((ch,), jnp.float32),
            pltpu.SemaphoreType.DMA,
            pltpu.SemaphoreType.DMA,
        ],
        compiler_params=pltpu.CompilerParams(use_tc_tiling_on_sc=False),
    )
    def sc_gather(emb_hbm, lin_hbm, idx_hbm, emb_out, lin_out,
                  idx_v, rows_v, lin_v, sem_e, sem_l):
        wid = lax.axis_index("s") * num_cores + lax.axis_index("c")
        for i in range(n_chunks):
            rbase = wid * rows_w + i * chr_
            base = rbase * IDX_COLS
            pltpu.sync_copy(idx_hbm.at[pl.ds(rbase, chr_)], idx_v)
            cps = []
            for j in range(chr_):
                cps.append(pltpu.async_copy(
                    emb_hbm.at[idx_v.at[j]],
                    rows_v.at[pl.ds(j * IDX_COLS, IDX_COLS)], sem_e))
                cps.append(pltpu.async_copy(
                    lin_hbm.at[idx_v.at[j]],
                    lin_v.at[pl.ds(j * IDX_COLS, IDX_COLS)], sem_l))
            for cp in cps:
                cp.wait()
            pltpu.sync_copy(rows_v, emb_out.at[pl.ds(base, ch)])
            pltpu.sync_copy(lin_v, lin_out.at[pl.ds(base, ch)])

    return sc_gather


def _mlp_body(emb_ref, lin_ref, w1_ref, b1_ref, w2_ref, b2_ref, w3_ref,
              bias_ref, out_ref):
    e = emb_ref[...]
    h = jnp.dot(e, w1_ref[...], preferred_element_type=jnp.float32)
    h = jnp.maximum(h + b1_ref[...], 0.0)
    h = jnp.dot(h, w2_ref[...], preferred_element_type=jnp.float32)
    h = jnp.maximum(h + b2_ref[...], 0.0)
    deep = jnp.dot(h, w3_ref[...], preferred_element_type=jnp.float32)
    wide = jnp.sum(lin_ref[...], axis=1, keepdims=True)
    z = deep + wide + bias_ref[...]
    out_ref[...] = 1.0 / (1.0 + jnp.exp(-z))


def _mlp(emb_x, lin_x, w1, b1, w2, b2, w3, bias):
    bt = 2048
    in_dim = NUM_FIELDS * EMBED_DIM
    h1, h2 = w1.shape[1], w2.shape[1]
    grid = (BATCH // bt,)
    return pl.pallas_call(
        _mlp_body,
        grid=grid,
        in_specs=[
            pl.BlockSpec((bt, in_dim), lambda i: (i, 0)),
            pl.BlockSpec((bt, NUM_FIELDS), lambda i: (i, 0)),
            pl.BlockSpec((in_dim, h1), lambda i: (0, 0)),
            pl.BlockSpec((1, h1), lambda i: (0, 0)),
            pl.BlockSpec((h1, h2), lambda i: (0, 0)),
            pl.BlockSpec((1, h2), lambda i: (0, 0)),
            pl.BlockSpec((h2, 1), lambda i: (0, 0)),
            pl.BlockSpec((1, 1), lambda i: (0, 0)),
        ],
        out_specs=pl.BlockSpec((bt, 1), lambda i: (i, 0)),
        out_shape=jax.ShapeDtypeStruct((BATCH, 1), jnp.float32),
    )(emb_x, lin_x, w1, b1, w2, b2, w3, bias)


def kernel(x, emb_table, lin_w, lin_b, w1, b1, w2, b2, w3, b3):
    offsets = (jnp.arange(NUM_FIELDS, dtype=x.dtype) * FIELD_SIZE)[None, :]
    idx = (x + offsets).reshape(IDX_ROWS, IDX_COLS)

    info = plsc.get_sparse_core_info()
    sc_gather = _make_sc_gather(info.num_cores, info.num_subcores)
    emb_rows, lin_rows = sc_gather(emb_table, lin_w.reshape(-1), idx)

    emb_x = emb_rows.reshape(BATCH, NUM_FIELDS * EMBED_DIM)
    lin_x = lin_rows.reshape(BATCH, NUM_FIELDS)

    bias = (lin_b + b3).reshape(1, 1)
    out = _mlp(emb_x, lin_x, w1, b1.reshape(1, -1), w2, b2.reshape(1, -1),
               w3, bias)
    return out[:, 0]


# speedup vs baseline: 7.3873x; 1.0170x over previous
import functools

import jax
import jax.numpy as jnp
from jax import lax
from jax.experimental import pallas as pl
from jax.experimental.pallas import tpu as pltpu
from jax.experimental.pallas import tpu_sc as plsc

FIELD_SIZE = 100000
NUM_FIELDS = 26
EMBED_DIM = 16
BATCH = 16384
TOTAL_IDX = BATCH * NUM_FIELDS


IDX_COLS = 128
IDX_ROWS = TOTAL_IDX // IDX_COLS
SLAB_W = 128
ROWS_PER_SLAB = SLAB_W // EMBED_DIM
SLAB_ROWS = FIELD_SIZE * NUM_FIELDS // ROWS_PER_SLAB
L = 16


@functools.cache
def _make_sc_emb_gather(num_cores, num_subcores):
    nw = num_cores * num_subcores
    rows_w = IDX_ROWS // nw
    chr_ = 13
    ch = chr_ * IDX_COLS
    n_chunks = rows_w // chr_
    mesh = plsc.VectorSubcoreMesh(core_axis_name="c", subcore_axis_name="s")

    @functools.partial(
        pl.kernel,
        mesh=mesh,
        out_type=jax.ShapeDtypeStruct((TOTAL_IDX, EMBED_DIM), jnp.float32),
        scratch_types=[
            pltpu.VMEM((chr_, IDX_COLS), jnp.int32),
            pltpu.VMEM((ch, EMBED_DIM), jnp.float32),
            pltpu.SemaphoreType.DMA,
        ],
        compiler_params=pltpu.CompilerParams(use_tc_tiling_on_sc=False),
    )
    def sc_emb(table_hbm, idx_hbm, emb_out, idx_v, rows_v, sem):
        wid = lax.axis_index("s") * num_cores + lax.axis_index("c")

        def chunk_body(i, _):
            rbase = wid * rows_w + i * chr_
            pltpu.sync_copy(idx_hbm.at[pl.ds(rbase, chr_)], idx_v)
            cps = [
                pltpu.async_copy(
                    table_hbm.at[idx_v.at[r]],
                    rows_v.at[pl.ds(r * IDX_COLS, IDX_COLS)], sem)
                for r in range(chr_)
            ]
            for cp in cps:
                cp.wait()
            pltpu.sync_copy(rows_v, emb_out.at[pl.ds(rbase * IDX_COLS, ch)])
            return 0

        lax.fori_loop(0, n_chunks, chunk_body, 0, unroll=False)

    return sc_emb


@functools.cache
def _make_sc_lin_gather(num_cores, num_subcores):
    nw = num_cores * num_subcores
    rows_w = IDX_ROWS // nw
    n_chunks = 8
    chr_ = rows_w // n_chunks
    ch = chr_ * IDX_COLS
    mesh = plsc.VectorSubcoreMesh(core_axis_name="c", subcore_axis_name="s")

    @functools.partial(
        pl.kernel,
        mesh=mesh,
        out_type=jax.ShapeDtypeStruct((TOTAL_IDX,), jnp.float32),
        scratch_types=[
            pltpu.VMEM((chr_, IDX_COLS), jnp.int32),
            pltpu.VMEM((ch,), jnp.float32),
            pltpu.SemaphoreType.DMA,
        ],
        compiler_params=pltpu.CompilerParams(use_tc_tiling_on_sc=False),
    )
    def sc_lin(lin_hbm, idx_hbm, lin_out, idx_v, lin_v, sem):
        wid = lax.axis_index("s") * num_cores + lax.axis_index("c")
        for i in range(n_chunks):
            rbase = wid * rows_w + i * chr_
            pltpu.sync_copy(idx_hbm.at[pl.ds(rbase, chr_)], idx_v)
            cps = [
                pltpu.async_copy(
                    lin_hbm.at[idx_v.at[j]],
                    lin_v.at[pl.ds(j * IDX_COLS, IDX_COLS)], sem)
                for j in range(chr_)
            ]
            for cp in cps:
                cp.wait()
            pltpu.sync_copy(lin_v, lin_out.at[pl.ds(rbase * IDX_COLS, ch)])

    return sc_lin


def _mlp_body(emb_ref, lin_ref, w1_ref, b1_ref, w2_ref, b2_ref, w3_ref,
              bias_ref, out_ref):
    e = emb_ref[...]
    h = jnp.dot(e, w1_ref[...], preferred_element_type=jnp.float32)
    h = jnp.maximum(h + b1_ref[...], 0.0)
    h = jnp.dot(h, w2_ref[...], preferred_element_type=jnp.float32)
    h = jnp.maximum(h + b2_ref[...], 0.0)
    deep = jnp.dot(h, w3_ref[...], preferred_element_type=jnp.float32)
    wide = jnp.sum(lin_ref[...], axis=1, keepdims=True)
    z = deep + wide + bias_ref[...]
    out_ref[...] = 1.0 / (1.0 + jnp.exp(-z))


def _mlp(emb_x, lin_x, w1, b1, w2, b2, w3, bias):
    bt = 2048
    in_dim = NUM_FIELDS * EMBED_DIM
    h1, h2 = w1.shape[1], w2.shape[1]
    grid = (BATCH // bt,)
    return pl.pallas_call(
        _mlp_body,
        grid=grid,
        in_specs=[
            pl.BlockSpec((bt, in_dim), lambda i: (i, 0)),
            pl.BlockSpec((bt, NUM_FIELDS), lambda i: (i, 0)),
            pl.BlockSpec((in_dim, h1), lambda i: (0, 0)),
            pl.BlockSpec((1, h1), lambda i: (0, 0)),
            pl.BlockSpec((h1, h2), lambda i: (0, 0)),
            pl.BlockSpec((1, h2), lambda i: (0, 0)),
            pl.BlockSpec((h2, 1), lambda i: (0, 0)),
            pl.BlockSpec((1, 1), lambda i: (0, 0)),
        ],
        out_specs=pl.BlockSpec((bt, 1), lambda i: (i, 0)),
        out_shape=jax.ShapeDtypeStruct((BATCH, 1), jnp.float32),
    )(emb_x, lin_x, w1, b1, w2, b2, w3, bias)


def kernel(x, emb_table, lin_w, lin_b, w1, b1, w2, b2, w3, b3):
    offsets = (jnp.arange(NUM_FIELDS, dtype=x.dtype) * FIELD_SIZE)[None, :]
    idx = (x + offsets).reshape(IDX_ROWS, IDX_COLS)

    info = plsc.get_sparse_core_info()
    sc_emb = _make_sc_emb_gather(info.num_cores, info.num_subcores)
    sc_lin = _make_sc_lin_gather(info.num_cores, info.num_subcores)
    emb_rows = sc_emb(emb_table, idx)
    lin_rows = sc_lin(lin_w[:, 0], idx)

    emb_x = emb_rows.reshape(BATCH, NUM_FIELDS * EMBED_DIM)
    lin_x = lin_rows.reshape(BATCH, NUM_FIELDS)

    bias = (lin_b + b3).reshape(1, 1)
    out = _mlp(emb_x, lin_x, w1, b1.reshape(1, -1), w2, b2.reshape(1, -1),
               w3, bias)
    return out[:, 0]


# speedup vs baseline: 7.3886x; 1.0002x over previous
import functools

import jax
import jax.numpy as jnp
from jax import lax
from jax.experimental import pallas as pl
from jax.experimental.pallas import tpu as pltpu
from jax.experimental.pallas import tpu_sc as plsc

FIELD_SIZE = 100000
NUM_FIELDS = 26
EMBED_DIM = 16
BATCH = 16384
TOTAL_IDX = BATCH * NUM_FIELDS


IDX_COLS = 128
IDX_ROWS = TOTAL_IDX // IDX_COLS


@functools.cache
def _make_sc_emb_gather(num_cores, num_subcores):
    nw = num_cores * num_subcores
    rows_w = IDX_ROWS // nw
    chr_ = 13
    ch = chr_ * IDX_COLS
    n_chunks = rows_w // chr_
    mesh = plsc.VectorSubcoreMesh(core_axis_name="c", subcore_axis_name="s")

    @functools.partial(
        pl.kernel,
        mesh=mesh,
        out_type=jax.ShapeDtypeStruct((TOTAL_IDX, EMBED_DIM), jnp.float32),
        scratch_types=[
            pltpu.VMEM((chr_, IDX_COLS), jnp.int32),
            pltpu.VMEM((ch, EMBED_DIM), jnp.float32),
            pltpu.SemaphoreType.DMA,
        ],
        compiler_params=pltpu.CompilerParams(use_tc_tiling_on_sc=False),
    )
    def sc_emb(table_hbm, idx_hbm, emb_out, idx_v, rows_v, sem):
        wid = lax.axis_index("s") * num_cores + lax.axis_index("c")

        def chunk_body(i, _):
            rbase = wid * rows_w + i * chr_
            pltpu.sync_copy(idx_hbm.at[pl.ds(rbase, chr_)], idx_v)
            cps = [
                pltpu.async_copy(
                    table_hbm.at[idx_v.at[r]],
                    rows_v.at[pl.ds(r * IDX_COLS, IDX_COLS)], sem)
                for r in range(chr_)
            ]
            for cp in cps:
                cp.wait()
            pltpu.sync_copy(rows_v, emb_out.at[pl.ds(rbase * IDX_COLS, ch)])
            return 0

        lax.fori_loop(0, n_chunks, chunk_body, 0, unroll=False)

    return sc_emb


@functools.cache
def _make_sc_lin_gather(num_cores, num_subcores):
    nw = num_cores * num_subcores
    rows_w = IDX_ROWS // nw
    n_chunks = 8
    chr_ = rows_w // n_chunks
    ch = chr_ * IDX_COLS
    mesh = plsc.VectorSubcoreMesh(core_axis_name="c", subcore_axis_name="s")

    @functools.partial(
        pl.kernel,
        mesh=mesh,
        out_type=jax.ShapeDtypeStruct((TOTAL_IDX,), jnp.float32),
        scratch_types=[
            pltpu.VMEM((chr_, IDX_COLS), jnp.int32),
            pltpu.VMEM((ch,), jnp.float32),
            pltpu.SemaphoreType.DMA,
        ],
        compiler_params=pltpu.CompilerParams(use_tc_tiling_on_sc=False),
    )
    def sc_lin(lin_hbm, idx_hbm, lin_out, idx_v, lin_v, sem):
        wid = lax.axis_index("s") * num_cores + lax.axis_index("c")
        for i in range(n_chunks):
            rbase = wid * rows_w + i * chr_
            pltpu.sync_copy(idx_hbm.at[pl.ds(rbase, chr_)], idx_v)
            cps = [
                pltpu.async_copy(
                    lin_hbm.at[idx_v.at[j]],
                    lin_v.at[pl.ds(j * IDX_COLS, IDX_COLS)], sem)
                for j in range(chr_)
            ]
            for cp in cps:
                cp.wait()
            pltpu.sync_copy(lin_v, lin_out.at[pl.ds(rbase * IDX_COLS, ch)])

    return sc_lin


def _mlp_body(emb_ref, lin_ref, w1_ref, b1_ref, w2_ref, b2_ref, w3_ref,
              bias_ref, out_ref):
    e = emb_ref[...]
    h = jnp.dot(e, w1_ref[...], preferred_element_type=jnp.float32)
    h = jnp.maximum(h + b1_ref[...], 0.0)
    h = jnp.dot(h, w2_ref[...], preferred_element_type=jnp.float32)
    h = jnp.maximum(h + b2_ref[...], 0.0)
    deep = jnp.dot(h, w3_ref[...], preferred_element_type=jnp.float32)
    wide = jnp.sum(lin_ref[...], axis=1, keepdims=True)
    z = deep + wide + bias_ref[...]
    out_ref[...] = 1.0 / (1.0 + jnp.exp(-z))


def _mlp(emb_x, lin_x, w1, b1, w2, b2, w3, bias):
    bt = 2048
    in_dim = NUM_FIELDS * EMBED_DIM
    h1, h2 = w1.shape[1], w2.shape[1]
    grid = (BATCH // bt,)
    return pl.pallas_call(
        _mlp_body,
        grid=grid,
        in_specs=[
            pl.BlockSpec((bt, in_dim), lambda i: (i, 0)),
            pl.BlockSpec((bt, NUM_FIELDS), lambda i: (i, 0)),
            pl.BlockSpec((in_dim, h1), lambda i: (0, 0)),
            pl.BlockSpec((1, h1), lambda i: (0, 0)),
            pl.BlockSpec((h1, h2), lambda i: (0, 0)),
            pl.BlockSpec((1, h2), lambda i: (0, 0)),
            pl.BlockSpec((h2, 1), lambda i: (0, 0)),
            pl.BlockSpec((1, 1), lambda i: (0, 0)),
        ],
        out_specs=pl.BlockSpec((bt, 1), lambda i: (i, 0)),
        out_shape=jax.ShapeDtypeStruct((BATCH, 1), jnp.float32),
    )(emb_x, lin_x, w1, b1, w2, b2, w3, bias)


def kernel(x, emb_table, lin_w, lin_b, w1, b1, w2, b2, w3, b3):
    offsets = (jnp.arange(NUM_FIELDS, dtype=x.dtype) * FIELD_SIZE)[None, :]
    idx = (x + offsets).reshape(IDX_ROWS, IDX_COLS)

    info = plsc.get_sparse_core_info()
    sc_emb = _make_sc_emb_gather(info.num_cores, info.num_subcores)
    sc_lin = _make_sc_lin_gather(info.num_cores, info.num_subcores)
    emb_rows = sc_emb(emb_table, idx)
    lin_rows = sc_lin(lin_w[:, 0], idx)

    emb_x = emb_rows.reshape(BATCH, NUM_FIELDS * EMBED_DIM)
    lin_x = lin_rows.reshape(BATCH, NUM_FIELDS)

    bias = (lin_b + b3).reshape(1, 1)
    out = _mlp(emb_x, lin_x, w1, b1.reshape(1, -1), w2, b2.reshape(1, -1),
               w3, bias)
    return out[:, 0]
